# baseline (speedup 1.0000x reference)
import functools

import jax
import jax.numpy as jnp
import numpy as np
from jax import lax
from jax.experimental import pallas as pl
from jax.experimental.pallas import tpu as pltpu
from jax.experimental.pallas import tpu_sc as plsc

_B, _S, _D = 1, 2048, 1024
_E, _K = 8, 2
_NWR = 16
_TPW = _S // _NWR
_NV = _TPW // 16
_CPW = _D // _NWR


def _noise_t():
    g = jax.random.gumbel(jax.random.PRNGKey(0), (_B, _S, _E), jnp.float32)
    return (g * 0.05).reshape(_S, _E).T


def _logits_kernel(x_ref, wr_ref, noise_ref, out_ref):
    logits = jnp.dot(x_ref[...], wr_ref[...],
                     preferred_element_type=jnp.float32)
    out_ref[...] = logits.T + noise_ref[...]


def _route_body(ln_hbm, x8_hbm, stage_hbm, out_hbm, lnv, wrow, pwv, x8v, ov):
    cid = lax.axis_index("c")
    sid = lax.axis_index("s")
    on_core0 = cid == 0

    @pl.when(on_core0)
    def _route():
        base = sid * _TPW
        for e in range(_E):
            pltpu.sync_copy(ln_hbm.at[e, pl.ds(base, _TPW)],
                            lnv.at[pl.ds(e * _TPW, _TPW)])

        acc = [jnp.zeros((16,), jnp.float32) for _ in range(2 * _E)]
        for j in range(_NV):
            lv = [lnv[pl.ds(e * _TPW + 16 * j, 16)] for e in range(_E)]
            m = lv[0]
            for e in range(1, _E):
                m = jnp.maximum(m, lv[e])
            p = [jnp.exp(lv[e] - m) for e in range(_E)]
            denom = p[0]
            for e in range(1, _E):
                denom = denom + p[e]
            probs = [p[e] / denom for e in range(_E)]
            g1 = probs[0]
            i1 = jnp.zeros((16,), jnp.int32)
            for e in range(1, _E):
                gt = probs[e] > g1
                g1 = jnp.where(gt, probs[e], g1)
                i1 = jnp.where(gt, e, i1)
            g2 = jnp.full((16,), -1.0, jnp.float32)
            i2 = jnp.zeros((16,), jnp.int32)
            for e in range(_E):
                cand = jnp.where(i1 == e, -1.0, probs[e])
                gt = cand > g2
                g2 = jnp.where(gt, cand, g2)
                i2 = jnp.where(gt, e, i2)
            for e in range(_E):
                acc[e] = acc[e] + jnp.where(i1 == e, g1, 0.0)
                acc[_E + e] = acc[_E + e] + jnp.where(i2 == e, g2, 0.0)

        for i in range(2 * _E):
            wrow[pl.ds(16 * i, 16)] = acc[i]
        pltpu.sync_copy(wrow, stage_hbm.at[pl.ds(sid * 256, 256)])

    plsc.subcore_barrier()

    @pl.when(on_core0)
    def _combine():
        pltpu.sync_copy(stage_hbm, pwv)
        for i in range(2 * _E):
            v = pwv[pl.ds(i * 16, 16)]
            for w in range(1, _NWR):
                v = v + pwv[pl.ds(w * 256 + i * 16, 16)]
            wrow[pl.ds(i * 16, 16)] = v
        wk = []
        for i in range(2 * _E):
            v = wrow[pl.ds(i * 16, 16)]
            s = v[0]
            for l in range(1, 16):
                s = s + v[l]
            wk.append(s)
        cbase = sid * _CPW
        for e in range(_E):
            pltpu.sync_copy(x8_hbm.at[e, pl.ds(cbase, _CPW)],
                            x8v.at[pl.ds(e * _CPW, _CPW)])
        for k in range(_K):
            for c in range(_CPW // 16):
                o = jnp.zeros((16,), jnp.float32)
                for e in range(_E):
                    o = o + wk[k * _E + e] * x8v[pl.ds(e * _CPW + 16 * c, 16)]
                ov[pl.ds(k * _CPW + 16 * c, 16)] = o
        for k in range(_K):
            pltpu.sync_copy(ov.at[pl.ds(k * _CPW, _CPW)],
                            out_hbm.at[k, pl.ds(cbase, _CPW)])


_route = functools.partial(
    pl.kernel,
    out_type=(
        jax.ShapeDtypeStruct((_NWR * 256,), jnp.float32),
        jax.ShapeDtypeStruct((_K, _D), jnp.float32),
    ),
    mesh=plsc.VectorSubcoreMesh(core_axis_name="c", subcore_axis_name="s"),
    scratch_types=[
        pltpu.VMEM((_E * _TPW,), jnp.float32),
        pltpu.VMEM((2 * _E * 16,), jnp.float32),
        pltpu.VMEM((_NWR * 256,), jnp.float32),
        pltpu.VMEM((_E * _CPW,), jnp.float32),
        pltpu.VMEM((_K * _CPW,), jnp.float32),
    ],
)(_route_body)


def kernel(inputs, w_router, W1, b1, W2, b2, WO, bO):
    del W1, b1, W2, b2, WO, bO
    x = inputs.reshape(_S, _D).astype(jnp.float32)

    ln = pl.pallas_call(
        _logits_kernel,
        out_shape=jax.ShapeDtypeStruct((_E, _S), jnp.float32),
    )(x, w_router.astype(jnp.float32), _noise_t())

    _, out = _route(ln, x[:_E])
    return out[None]


# speedup vs baseline: 1.2791x; 1.2791x over previous
import functools

import jax
import jax.numpy as jnp
import numpy as np
from jax import lax
from jax.experimental import pallas as pl
from jax.experimental.pallas import tpu as pltpu
from jax.experimental.pallas import tpu_sc as plsc

_B, _S, _D = 1, 2048, 1024
_E, _K = 8, 2
_NW = 16
_TPW = _S // _NW
_NV = _TPW // 16


def _noise_t():
    g = jax.random.gumbel(jax.random.PRNGKey(0), (_B, _S, _E), jnp.float32)
    return (g * 0.05).reshape(_S, _E).T


def _logits_kernel(x_ref, wr_ref, noise_ref, out_ref):
    logits = jnp.dot(x_ref[...], wr_ref[...],
                     preferred_element_type=jnp.float32)
    out_ref[...] = logits.T + noise_ref[...]


def _route_body(ln_hbm, out_hbm, lnv, wrow):
    wid = lax.axis_index("s")
    base = wid * _TPW
    for e in range(_E):
        pltpu.sync_copy(ln_hbm.at[e, pl.ds(base, _TPW)],
                        lnv.at[pl.ds(e * _TPW, _TPW)])

    acc = [jnp.zeros((16,), jnp.float32) for _ in range(2 * _E)]
    for j in range(_NV):
        lv = [lnv[pl.ds(e * _TPW + 16 * j, 16)] for e in range(_E)]
        m = lv[0]
        for e in range(1, _E):
            m = jnp.maximum(m, lv[e])
        p = [jnp.exp(lv[e] - m) for e in range(_E)]
        denom = p[0]
        for e in range(1, _E):
            denom = denom + p[e]
        probs = [p[e] / denom for e in range(_E)]
        g1 = probs[0]
        i1 = jnp.zeros((16,), jnp.int32)
        for e in range(1, _E):
            gt = probs[e] > g1
            g1 = jnp.where(gt, probs[e], g1)
            i1 = jnp.where(gt, e, i1)
        g2 = jnp.full((16,), -1.0, jnp.float32)
        i2 = jnp.zeros((16,), jnp.int32)
        for e in range(_E):
            cand = jnp.where(i1 == e, -1.0, probs[e])
            gt = cand > g2
            g2 = jnp.where(gt, cand, g2)
            i2 = jnp.where(gt, e, i2)
        for e in range(_E):
            acc[e] = acc[e] + jnp.where(i1 == e, g1, 0.0)
            acc[_E + e] = acc[_E + e] + jnp.where(i2 == e, g2, 0.0)

    for i in range(2 * _E):
        wrow[pl.ds(16 * i, 16)] = acc[i]
    pltpu.sync_copy(wrow, out_hbm.at[wid])


_route = functools.partial(
    pl.kernel,
    out_type=jax.ShapeDtypeStruct((_NW, 2 * _E * 16), jnp.float32),
    mesh=plsc.VectorSubcoreMesh(core_axis_name="c", subcore_axis_name="s",
                                num_cores=1),
    scratch_types=[
        pltpu.VMEM((_E * _TPW,), jnp.float32),
        pltpu.VMEM((2 * _E * 16,), jnp.float32),
    ],
)(_route_body)


def _combine_kernel(pw_ref, x8_ref, out_ref):
    s = jnp.sum(pw_ref[...], axis=0, keepdims=True)
    x8 = x8_ref[...]
    for k in range(_K):
        o = jnp.zeros((1, _D), jnp.float32)
        for e in range(_E):
            base = (_E * k + e) * 16
            scal = jnp.sum(s[0:1, base:base + 16])
            o = o + scal * x8[e:e + 1, :]
        out_ref[k:k + 1, :] = o


def kernel(inputs, w_router, W1, b1, W2, b2, WO, bO):
    del W1, b1, W2, b2, WO, bO
    x = inputs.reshape(_S, _D).astype(jnp.float32)

    ln = pl.pallas_call(
        _logits_kernel,
        out_shape=jax.ShapeDtypeStruct((_E, _S), jnp.float32),
    )(x, w_router.astype(jnp.float32), _noise_t())

    partials = _route(ln)

    out = pl.pallas_call(
        _combine_kernel,
        out_shape=jax.ShapeDtypeStruct((_K, _D), jnp.float32),
    )(partials, x[:_E])
    return out[None]


# speedup vs baseline: 1.4257x; 1.1146x over previous
import functools

import jax
import jax.numpy as jnp
import numpy as np
from jax import lax
from jax.experimental import pallas as pl
from jax.experimental.pallas import tpu as pltpu
from jax.experimental.pallas import tpu_sc as plsc

_B, _S, _D = 1, 2048, 1024
_E, _K = 8, 2
_NW = 16
_TPW = _S // _NW
_NV = _TPW // 16


def _noise_t():
    g = jax.random.gumbel(jax.random.PRNGKey(0), (_B, _S, _E), jnp.float32)
    return (g * 0.05).reshape(_S, _E).T


def _logits_kernel(x_ref, wr_ref, noise_ref, out_ref):
    logits = jnp.dot(x_ref[...], wr_ref[...],
                     preferred_element_type=jnp.float32)
    out_ref[...] = logits.T + noise_ref[...]


def _route_body(ln_hbm, out_hbm, lnv, wrow, sem):
    wid = lax.axis_index("s")
    base = wid * _TPW
    copies = [
        pltpu.async_copy(ln_hbm.at[e, pl.ds(base, _TPW)],
                         lnv.at[pl.ds(e * _TPW, _TPW)], sem)
        for e in range(_E)
    ]
    for c in copies:
        c.wait()

    acc = [jnp.zeros((16,), jnp.float32) for _ in range(2 * _E)]
    for j in range(_NV):
        lv = [lnv[pl.ds(e * _TPW + 16 * j, 16)] for e in range(_E)]
        m = lv[0]
        for e in range(1, _E):
            m = jnp.maximum(m, lv[e])
        p = [jnp.exp(lv[e] - m) for e in range(_E)]
        denom = p[0]
        for e in range(1, _E):
            denom = denom + p[e]
        probs = [p[e] / denom for e in range(_E)]
        g1 = probs[0]
        i1 = jnp.zeros((16,), jnp.int32)
        for e in range(1, _E):
            gt = probs[e] > g1
            g1 = jnp.where(gt, probs[e], g1)
            i1 = jnp.where(gt, e, i1)
        g2 = jnp.full((16,), -1.0, jnp.float32)
        i2 = jnp.zeros((16,), jnp.int32)
        for e in range(_E):
            cand = jnp.where(i1 == e, -1.0, probs[e])
            gt = cand > g2
            g2 = jnp.where(gt, cand, g2)
            i2 = jnp.where(gt, e, i2)
        for e in range(_E):
            acc[e] = acc[e] + jnp.where(i1 == e, g1, 0.0)
            acc[_E + e] = acc[_E + e] + jnp.where(i2 == e, g2, 0.0)

    for i in range(2 * _E):
        wrow[pl.ds(16 * i, 16)] = acc[i]
    pltpu.sync_copy(wrow, out_hbm.at[wid])


_route = functools.partial(
    pl.kernel,
    out_type=jax.ShapeDtypeStruct((_NW, 2 * _E * 16), jnp.float32),
    mesh=plsc.VectorSubcoreMesh(core_axis_name="c", subcore_axis_name="s",
                                num_cores=1),
    scratch_types=[
        pltpu.VMEM((_E * _TPW,), jnp.float32),
        pltpu.VMEM((2 * _E * 16,), jnp.float32),
        pltpu.SemaphoreType.DMA,
    ],
)(_route_body)


def _combine_kernel(pw_ref, x8_ref, out_ref):
    s = jnp.sum(pw_ref[...], axis=0, keepdims=True)
    x8 = x8_ref[...]
    for k in range(_K):
        o = jnp.zeros((1, _D), jnp.float32)
        for e in range(_E):
            base = (_E * k + e) * 16
            scal = jnp.sum(s[0:1, base:base + 16])
            o = o + scal * x8[e:e + 1, :]
        out_ref[k:k + 1, :] = o


def kernel(inputs, w_router, W1, b1, W2, b2, WO, bO):
    del W1, b1, W2, b2, WO, bO
    x = inputs.reshape(_S, _D).astype(jnp.float32)

    ln = pl.pallas_call(
        _logits_kernel,
        out_shape=jax.ShapeDtypeStruct((_E, _S), jnp.float32),
    )(x, w_router.astype(jnp.float32), _noise_t())

    partials = _route(ln)

    out = pl.pallas_call(
        _combine_kernel,
        out_shape=jax.ShapeDtypeStruct((_K, _D), jnp.float32),
    )(partials, x[:_E])
    return out[None]
